# baseline (speedup 1.0000x reference)
import functools

import jax
import jax.numpy as jnp
from jax import lax
from jax.experimental import pallas as pl
from jax.experimental.pallas import tpu as pltpu
from jax.experimental.pallas import tpu_sc as plsc

_CHUNK = 16
_NBUF = 3
_STREAM_CHUNKS = 9
_DMA_CHUNKS = 7


class _Pipe:
    def __init__(self, in_copy, out_copy, n_chunks, nbuf):
        self.in_copy, self.out_copy = in_copy, out_copy
        self.n, self.nbuf = n_chunks, nbuf

    def prime(self):
        for i in range(min(self.nbuf, self.n)):
            self.in_copy(i, i).start()

    def step(self, i):
        b = i % self.nbuf
        self.in_copy(i, b).wait()
        self.out_copy(i, b).start()
        nxt = i + self.nbuf
        if nxt < self.n:
            self.out_copy(i, b).wait()
            self.in_copy(nxt, b).start()

    def drain(self):
        for i in range(max(self.n - self.nbuf, 0), self.n):
            self.out_copy(i, i % self.nbuf).wait()


def _make_copy_kernel(seq_len: int, n_embd: int):
    info = plsc.get_sparse_core_info()
    nc, ns = info.num_cores, info.num_subcores
    nw = nc * ns
    rows_per_w = seq_len // nw
    assert rows_per_w == (_STREAM_CHUNKS + _DMA_CHUNKS) * _CHUNK
    mesh = plsc.VectorSubcoreMesh(core_axis_name="c", subcore_axis_name="s")

    @functools.partial(
        pl.kernel,
        mesh=mesh,
        out_type=jax.ShapeDtypeStruct((seq_len, n_embd), jnp.float32),
        scratch_types=[
            pltpu.VMEM((_NBUF, _CHUNK, n_embd), jnp.float32),
            pltpu.VMEM_SHARED((ns, _NBUF, _CHUNK, n_embd), jnp.float32),
            pltpu.SemaphoreType.DMA((_NBUF,)),
            pltpu.SemaphoreType.DMA((_NBUF,)),
            pltpu.SemaphoreType.DMA((_NBUF,)),
            pltpu.SemaphoreType.DMA((_NBUF,)),
        ],
    )
    def copy_kernel(emb_hbm, out_hbm, buf, buf_sh, s_in, s_out, d_in, d_out):
        sid = lax.axis_index("s")
        wid = sid * nc + lax.axis_index("c")
        base = wid * rows_per_w
        d_base = base + _STREAM_CHUNKS * _CHUNK

        def s_in_copy(i, b):
            return pltpu.make_async_copy(
                emb_hbm.at[pl.ds(base + i * _CHUNK, _CHUNK)],
                buf.at[b], s_in.at[b])

        def s_out_copy(i, b):
            return pltpu.make_async_copy(
                buf.at[b],
                out_hbm.at[pl.ds(base + i * _CHUNK, _CHUNK)], s_out.at[b])

        def d_in_copy(i, b):
            return pltpu.make_async_copy(
                emb_hbm.at[pl.ds(d_base + i * _CHUNK, _CHUNK)],
                buf_sh.at[sid, b], d_in.at[b])

        def d_out_copy(i, b):
            return pltpu.make_async_copy(
                buf_sh.at[sid, b],
                out_hbm.at[pl.ds(d_base + i * _CHUNK, _CHUNK)], d_out.at[b])

        sp = _Pipe(s_in_copy, s_out_copy, _STREAM_CHUNKS, _NBUF)
        dp = _Pipe(d_in_copy, d_out_copy, _DMA_CHUNKS, _NBUF)
        sp.prime()
        dp.prime()
        for i in range(max(_STREAM_CHUNKS, _DMA_CHUNKS)):
            if i < _STREAM_CHUNKS:
                sp.step(i)
            if i < _DMA_CHUNKS:
                dp.step(i)
        sp.drain()
        dp.drain()

    return copy_kernel


def kernel(x, emb):
    seq_len = x.shape[1]
    return _make_copy_kernel(seq_len, emb.shape[1])(emb)
